# baseline (speedup 1.0000x reference)
import functools

import jax
import jax.numpy as jnp
from jax import lax
from jax.experimental import pallas as pl
from jax.experimental.pallas import tpu as pltpu
from jax.experimental.pallas import tpu_sc as plsc

_E = 64
_K = 8
_T = 16384
_L = 16
_NC = 2
_NS = 16
_NW = _NC * _NS
_TPW = _T // _NW
_G = _TPW // _L
_S = 8
_SW = _E // _S
_TB = _TPW // 128

_MESH = plsc.VectorSubcoreMesh(core_axis_name="c", subcore_axis_name="s")


def _splat_i32(x):
    return jnp.full((_L,), x, dtype=jnp.int32)


def _argmax_tree(vals, idxs):
    vals = list(vals)
    idxs = list(idxs)
    while len(vals) > 1:
        nv, ni = [], []
        for i in range(0, len(vals), 2):
            gt = vals[i + 1] > vals[i]
            nv.append(jnp.where(gt, vals[i + 1], vals[i]))
            ni.append(jnp.where(gt, idxs[i + 1], idxs[i]))
        vals, idxs = nv, ni
    return vals[0], idxs[0]


@functools.partial(
    pl.kernel,
    out_type=(
        jax.ShapeDtypeStruct((_T * _K,), jnp.int32),
        jax.ShapeDtypeStruct((_T * _K,), jnp.float32),
    ),
    mesh=_MESH,
    scratch_types=[
        pltpu.VMEM((_TPW * _E,), jnp.float32),
        pltpu.VMEM((_TPW * _K,), jnp.int32),
        pltpu.VMEM((_TPW * _K,), jnp.float32),
        pltpu.SemaphoreType.DMA,
        pltpu.SemaphoreType.DMA,
        pltpu.SemaphoreType.DMA,
        pltpu.SemaphoreType.DMA,
        pltpu.SemaphoreType.DMA,
    ],
    compiler_params=pltpu.CompilerParams(needs_layout_passes=False),
)
def _select_topk_sc(logits_hbm, ids_hbm, w_hbm, lv, idv, wv,
                    sem0, sem1, sem2, sem3, osem):
    wid = lax.axis_index("s") * _NC + lax.axis_index("c")
    sems = [sem0, sem1, sem2, sem3]

    col_descs = []
    for c in range(_TB):
        ds = []
        for eb in range(_S):
            src = logits_hbm.at[pl.ds((eb * (_T // 128) + wid * _TB + c)
                                      * 1024, 1024)]
            dst = lv.at[pl.ds((eb * _TB + c) * 1024, 1024)]
            ds.append(pltpu.async_copy(src, dst, sems[c]))
        col_descs.append(ds)
    for d in col_descs[0]:
        d.wait()

    lane = lax.iota(jnp.int32, _L)
    neg_inf = jnp.full((_L,), -jnp.inf, dtype=jnp.float32)

    _P = 2

    def pair_body(p, carry):
        for c in range(1, _TB):
            @pl.when(p == c * (8 // _P))
            def _():
                for d in col_descs[c]:
                    d.wait()
        st = []
        for gg in range(_P):
            g = p * _P + gg
            gbase = ((g >> 3) * 1024) + ((g * _L) & 127)
            st.append({"gbase": gbase, "gv": gbase + lane})

        for s in st:
            gbase = s["gbase"]
            segmax, segarg = [], []
            for si in range(_S):
                elems = [lv[pl.ds(si * 4096 + k * 128 + gbase, _L)]
                         for k in range(_SW)]
                cols = [_splat_i32(si * _SW + k) for k in range(_SW)]
                m, bi = _argmax_tree(elems, cols)
                segmax.append(m)
                segarg.append(bi)
            s["segmax"], s["segarg"] = segmax, segarg
            s["exps"] = []

        for r in range(_K):
            for s in st:
                gm, ga = _argmax_tree(s["segmax"], s["segarg"])
                idv[pl.ds(s["gbase"] + r * 128, _L)] = ga
                if r == 0:
                    s["v0"] = gm
                    s["exps"].append(jnp.full((_L,), 1.0, jnp.float32))
                else:
                    s["exps"].append(jnp.exp(gm - s["v0"]))
                s["ga"] = ga
            if r == _K - 1:
                break
            for s in st:
                ga = s["ga"]
                sv = ((ga >> 3) << 12) + s["gv"]
                eb = ga & _splat_i32(-_SW)
                plsc.store_scatter(lv, [sv + ((ga & _splat_i32(7)) << 7)],
                                   neg_inf)
                elems = [plsc.load_gather(lv, [sv + _splat_i32(k * 128)])
                         for k in range(_SW)]
                cols = [eb + _splat_i32(k) for k in range(_SW)]
                nm, nb = _argmax_tree(elems, cols)
                for si in range(_S):
                    hit = eb == _splat_i32(si * _SW)
                    s["segmax"][si] = jnp.where(hit, nm, s["segmax"][si])
                    s["segarg"][si] = jnp.where(hit, nb, s["segarg"][si])

        for s in st:
            exps = s["exps"]
            tot = exps[0]
            for r in range(1, _K):
                tot = tot + exps[r]
            inv = jnp.float32(1.0) / tot
            for r in range(_K):
                wv[pl.ds(s["gbase"] + r * 128, _L)] = exps[r] * inv
        return carry

    lax.fori_loop(0, _G // _P, pair_body, 0)

    oid = pltpu.async_copy(idv, ids_hbm.at[pl.ds(wid * _TB * 1024,
                                                 _TB * 1024)], osem)
    ow = pltpu.async_copy(wv, w_hbm.at[pl.ds(wid * _TB * 1024,
                                             _TB * 1024)], osem)
    oid.wait()
    ow.wait()


def kernel(router_logits_fp32, topk_ids, topk_weights):
    x4 = jnp.transpose(router_logits_fp32).reshape(_S, _SW, _T // 128, 128)
    xf = jnp.transpose(x4, (0, 2, 1, 3)).reshape(-1)
    ids_f, w_f = _select_topk_sc(xf)
    ids = jnp.transpose(
        jnp.transpose(ids_f.reshape(_T // 128, _K, 128), (1, 0, 2))
        .reshape(_K, _T))
    w = jnp.transpose(
        jnp.transpose(w_f.reshape(_T // 128, _K, 128), (1, 0, 2))
        .reshape(_K, _T))
    return ids, w


# speedup vs baseline: 1.0214x; 1.0214x over previous
import functools

import jax
import jax.numpy as jnp
from jax import lax
from jax.experimental import pallas as pl
from jax.experimental.pallas import tpu as pltpu
from jax.experimental.pallas import tpu_sc as plsc

_E = 64
_K = 8
_T = 16384
_L = 16
_NC = 2
_NS = 16
_NW = _NC * _NS
_TPW = _T // _NW
_G = _TPW // _L
_S = 8
_SW = _E // _S
_TB = _TPW // 128

_MESH = plsc.VectorSubcoreMesh(core_axis_name="c", subcore_axis_name="s")


def _splat_i32(x):
    return jnp.full((_L,), x, dtype=jnp.int32)


def _argmax_tree(vals, idxs):
    vals = list(vals)
    idxs = list(idxs)
    while len(vals) > 1:
        nv, ni = [], []
        for i in range(0, len(vals), 2):
            gt = vals[i + 1] > vals[i]
            nv.append(jnp.maximum(vals[i], vals[i + 1]))
            ni.append(jnp.where(gt, idxs[i + 1], idxs[i]))
        vals, idxs = nv, ni
    return vals[0], idxs[0]


@functools.partial(
    pl.kernel,
    out_type=(
        jax.ShapeDtypeStruct((_T * _K,), jnp.int32),
        jax.ShapeDtypeStruct((_T * _K,), jnp.float32),
    ),
    mesh=_MESH,
    scratch_types=[
        pltpu.VMEM((_TPW * _E,), jnp.float32),
        pltpu.VMEM((_TPW * _K,), jnp.int32),
        pltpu.VMEM((_TPW * _K,), jnp.float32),
        pltpu.SemaphoreType.DMA,
        pltpu.SemaphoreType.DMA,
        pltpu.SemaphoreType.DMA,
        pltpu.SemaphoreType.DMA,
        pltpu.SemaphoreType.DMA,
    ],
    compiler_params=pltpu.CompilerParams(needs_layout_passes=False),
)
def _select_topk_sc(logits_hbm, ids_hbm, w_hbm, lv, idv, wv,
                    sem0, sem1, sem2, sem3, osem):
    wid = lax.axis_index("s") * _NC + lax.axis_index("c")
    sems = [sem0, sem1, sem2, sem3]

    copies = []
    for eb in range(_S):
        src = logits_hbm.at[pl.ds((eb * (_T // 128) + wid * _TB) * 1024,
                                  _TB * 1024)]
        dst = lv.at[pl.ds(eb * _TB * 1024, _TB * 1024)]
        copies.append(pltpu.async_copy(src, dst, sems[eb % 4]))
    for c in copies:
        c.wait()

    lane = lax.iota(jnp.int32, _L)
    neg_inf = jnp.full((_L,), -jnp.inf, dtype=jnp.float32)

    _P = 2

    def pair_body(p, carry):
        st = []
        for gg in range(_P):
            g = p * _P + gg
            gbase = ((g >> 3) * 1024) + ((g * _L) & 127)
            st.append({"gbase": gbase, "gv": gbase + lane})

        for s in st:
            gbase = s["gbase"]
            segmax, segarg = [], []
            for si in range(_S):
                elems = [lv[pl.ds(si * 4096 + k * 128 + gbase, _L)]
                         for k in range(_SW)]
                cols = [_splat_i32(si * _SW + k) for k in range(_SW)]
                m, bi = _argmax_tree(elems, cols)
                segmax.append(m)
                segarg.append(bi)
            s["segmax"], s["segarg"] = segmax, segarg
            s["exps"] = []

        for r in range(_K):
            for s in st:
                gm, ga = _argmax_tree(s["segmax"], s["segarg"])
                idv[pl.ds(s["gbase"] + r * 128, _L)] = ga
                if r == 0:
                    s["v0"] = gm
                    s["exps"].append(jnp.full((_L,), 1.0, jnp.float32))
                else:
                    s["exps"].append(jnp.exp(gm - s["v0"]))
                s["ga"] = ga
            if r == _K - 1:
                break
            for s in st:
                ga = s["ga"]
                sv = ((ga >> 3) << 12) + s["gv"]
                eb = ga & _splat_i32(-_SW)
                plsc.store_scatter(lv, [sv + ((ga & _splat_i32(7)) << 7)],
                                   neg_inf)
                elems = [plsc.load_gather(lv, [sv + _splat_i32(k * 128)])
                         for k in range(_SW)]
                cols = [eb + _splat_i32(k) for k in range(_SW)]
                nm, nb = _argmax_tree(elems, cols)
                for si in range(_S):
                    hit = eb == _splat_i32(si * _SW)
                    s["segmax"][si] = jnp.where(hit, nm, s["segmax"][si])
                    s["segarg"][si] = jnp.where(hit, nb, s["segarg"][si])

        for s in st:
            exps = s["exps"]
            tot = exps[0]
            for r in range(1, _K):
                tot = tot + exps[r]
            inv = jnp.float32(1.0) / tot
            for r in range(_K):
                wv[pl.ds(s["gbase"] + r * 128, _L)] = exps[r] * inv
        return carry

    lax.fori_loop(0, _G // _P, pair_body, 0)

    oid = pltpu.async_copy(idv, ids_hbm.at[pl.ds(wid * _TB * 1024,
                                                 _TB * 1024)], osem)
    ow = pltpu.async_copy(wv, w_hbm.at[pl.ds(wid * _TB * 1024,
                                             _TB * 1024)], osem)
    oid.wait()
    ow.wait()


def kernel(router_logits_fp32, topk_ids, topk_weights):
    x4 = jnp.transpose(router_logits_fp32).reshape(_S, _SW, _T // 128, 128)
    xf = jnp.transpose(x4, (0, 2, 1, 3)).reshape(-1)
    ids_f, w_f = _select_topk_sc(xf)
    ids = jnp.transpose(
        jnp.transpose(ids_f.reshape(_T // 128, _K, 128), (1, 0, 2))
        .reshape(_K, _T))
    w = jnp.transpose(
        jnp.transpose(w_f.reshape(_T // 128, _K, 128), (1, 0, 2))
        .reshape(_K, _T))
    return ids, w


# speedup vs baseline: 1.0286x; 1.0071x over previous
import functools

import jax
import jax.numpy as jnp
from jax import lax
from jax.experimental import pallas as pl
from jax.experimental.pallas import tpu as pltpu
from jax.experimental.pallas import tpu_sc as plsc

_E = 64
_K = 8
_T = 16384
_L = 16
_NC = 2
_NS = 16
_NW = _NC * _NS
_TPW = _T // _NW
_G = _TPW // _L
_S = 8
_SW = _E // _S
_TB = _TPW // 128

_MESH = plsc.VectorSubcoreMesh(core_axis_name="c", subcore_axis_name="s")


def _splat_i32(x):
    return jnp.full((_L,), x, dtype=jnp.int32)


def _argmax_tree(vals, idxs):
    vals = list(vals)
    idxs = list(idxs)
    while len(vals) > 1:
        nv, ni = [], []
        for i in range(0, len(vals), 2):
            gt = vals[i + 1] > vals[i]
            nv.append(jnp.maximum(vals[i], vals[i + 1]))
            ni.append(jnp.where(gt, idxs[i + 1], idxs[i]))
        vals, idxs = nv, ni
    return vals[0], idxs[0]


@functools.partial(
    pl.kernel,
    out_type=(
        jax.ShapeDtypeStruct((_T * _K,), jnp.int32),
        jax.ShapeDtypeStruct((_T * _K,), jnp.float32),
    ),
    mesh=_MESH,
    scratch_types=[
        pltpu.VMEM((_TPW * _E,), jnp.float32),
        pltpu.VMEM((_TPW * _K,), jnp.int32),
        pltpu.VMEM((_TPW * _K,), jnp.float32),
        pltpu.SemaphoreType.DMA,
        pltpu.SemaphoreType.DMA,
        pltpu.SemaphoreType.DMA,
        pltpu.SemaphoreType.DMA,
        pltpu.SemaphoreType.DMA,
    ],
    compiler_params=pltpu.CompilerParams(needs_layout_passes=False),
)
def _select_topk_sc(logits_hbm, ids_hbm, w_hbm, lv, idv, wv,
                    sem0, sem1, sem2, sem3, osem):
    wid = lax.axis_index("s") * _NC + lax.axis_index("c")
    sems = [sem0, sem1, sem2, sem3]

    copies = []
    for eb in range(_S):
        src = logits_hbm.at[pl.ds((eb * (_T // 128) + wid * _TB) * 1024,
                                  _TB * 1024)]
        dst = lv.at[pl.ds(eb * _TB * 1024, _TB * 1024)]
        copies.append(pltpu.async_copy(src, dst, sems[eb % 4]))
    for c in copies:
        c.wait()

    lane = lax.iota(jnp.int32, _L)
    neg_inf = jnp.full((_L,), -jnp.inf, dtype=jnp.float32)

    _P = 2

    def pair_body(p, carry):
        st = []
        for gg in range(_P):
            g = p * _P + gg
            gbase = ((g >> 3) * 1024) + ((g * _L) & 127)
            st.append({"gbase": gbase, "gv": gbase + lane})

        for s in st:
            gbase = s["gbase"]
            segmax, segarg = [], []
            for si in range(_S):
                elems = [lv[pl.ds(si * 4096 + k * 128 + gbase, _L)]
                         for k in range(_SW)]
                cols = [_splat_i32(si * _SW + k) for k in range(_SW)]
                m, bi = _argmax_tree(elems, cols)
                segmax.append(m)
                segarg.append(bi)
            s["segmax"], s["segarg"] = segmax, segarg
            s["exps"] = []

        for r in range(_K):
            for s in st:
                gm, ga = _argmax_tree(s["segmax"], s["segarg"])
                idv[pl.ds(s["gbase"] + r * 128, _L)] = ga
                if r == 0:
                    s["v0"] = gm
                    s["exps"].append(jnp.full((_L,), 1.0, jnp.float32))
                else:
                    s["exps"].append(jnp.exp(gm - s["v0"]))
                s["ga"] = ga
            if r == _K - 1:
                break
            for s in st:
                ga = s["ga"]
                eb = ga & _splat_i32(-_SW)
                sv = (eb << 9) + s["gv"]
                plsc.store_scatter(lv, [sv + ((ga & _splat_i32(7)) << 7)],
                                   neg_inf)
                elems = [plsc.load_gather(lv, [sv + _splat_i32(k * 128)])
                         for k in range(_SW)]
                cols = [eb + _splat_i32(k) for k in range(_SW)]
                nm, nb = _argmax_tree(elems, cols)
                for si in range(_S):
                    hit = eb == _splat_i32(si * _SW)
                    s["segmax"][si] = jnp.where(hit, nm, s["segmax"][si])
                    s["segarg"][si] = jnp.where(hit, nb, s["segarg"][si])

        for s in st:
            exps = s["exps"]
            tot = exps[0]
            for r in range(1, _K):
                tot = tot + exps[r]
            inv = jnp.float32(1.0) / tot
            for r in range(_K):
                wv[pl.ds(s["gbase"] + r * 128, _L)] = exps[r] * inv
        return carry

    lax.fori_loop(0, _G // _P, pair_body, 0)

    oid = pltpu.async_copy(idv, ids_hbm.at[pl.ds(wid * _TB * 1024,
                                                 _TB * 1024)], osem)
    ow = pltpu.async_copy(wv, w_hbm.at[pl.ds(wid * _TB * 1024,
                                             _TB * 1024)], osem)
    oid.wait()
    ow.wait()


def kernel(router_logits_fp32, topk_ids, topk_weights):
    x4 = jnp.transpose(router_logits_fp32).reshape(_S, _SW, _T // 128, 128)
    xf = jnp.transpose(x4, (0, 2, 1, 3)).reshape(-1)
    ids_f, w_f = _select_topk_sc(xf)
    ids = jnp.transpose(
        jnp.transpose(ids_f.reshape(_T // 128, _K, 128), (1, 0, 2))
        .reshape(_K, _T))
    w = jnp.transpose(
        jnp.transpose(w_f.reshape(_T // 128, _K, 128), (1, 0, 2))
        .reshape(_K, _T))
    return ids, w
